# Initial kernel scaffold; baseline (speedup 1.0000x reference)
import jax
import jax.numpy as jnp
from jax.experimental import pallas as pl


def kernel(x, pos_emb_weight):
    raise NotImplementedError("write your pallas kernel here")


# speedup vs baseline: 1.2060x; 1.2060x over previous
import jax
import jax.numpy as jnp
from jax.experimental import pallas as pl

BLOCK_S = 512


def _add_kernel(x_ref, pe_ref, o_ref):
    o_ref[...] = x_ref[...] + pe_ref[...]


def kernel(x, pos_emb_weight):
    b, s, d = x.shape
    grid = (b, s // BLOCK_S)
    return pl.pallas_call(
        _add_kernel,
        grid=grid,
        in_specs=[
            pl.BlockSpec((1, BLOCK_S, d), lambda i, j: (i, j, 0)),
            pl.BlockSpec((BLOCK_S, d), lambda i, j: (j, 0)),
        ],
        out_specs=pl.BlockSpec((1, BLOCK_S, d), lambda i, j: (i, j, 0)),
        out_shape=jax.ShapeDtypeStruct((b, s, d), x.dtype),
    )(x, pos_emb_weight)


# speedup vs baseline: 1.5926x; 1.3205x over previous
import jax
import jax.numpy as jnp
from jax.experimental import pallas as pl

BLOCK_S = 512


def _add_kernel(x_ref, pe_ref, o_ref):
    o_ref[...] = x_ref[...] + pe_ref[...]


def kernel(x, pos_emb_weight):
    b, s, d = x.shape
    grid = (s // BLOCK_S, b)
    return pl.pallas_call(
        _add_kernel,
        grid=grid,
        in_specs=[
            pl.BlockSpec((1, BLOCK_S, d), lambda j, i: (i, j, 0)),
            pl.BlockSpec((BLOCK_S, d), lambda j, i: (j, 0)),
        ],
        out_specs=pl.BlockSpec((1, BLOCK_S, d), lambda j, i: (i, j, 0)),
        out_shape=jax.ShapeDtypeStruct((b, s, d), x.dtype),
    )(x, pos_emb_weight)
